# baseline (speedup 1.0000x reference)
import jax
import jax.numpy as jnp
from jax.experimental import pallas as pl

INPUT = 784
HIDDEN = 8192
OUT = 10
BLK = 1024


def _fused_kernel(x_ref, wt_ref, wsup_ref, b_ref, out_ref):
    i = pl.program_id(0)
    x2 = x_ref[...].reshape(1, INPUT)
    h = jax.lax.dot_general(
        x2, wt_ref[...],
        (((1,), (0,)), ((), ())),
        preferred_element_type=jnp.float32,
    )
    h = jnp.maximum(h, 0.0)
    part = jax.lax.dot_general(
        h, wsup_ref[:, pl.ds(i * BLK, BLK)],
        (((1,), (1,)), ((), ())),
        preferred_element_type=jnp.float32,
    ).reshape(OUT)

    @pl.when(i == 0)
    def _():
        out_ref[...] = b_ref[...] + part

    @pl.when(i != 0)
    def _():
        out_ref[...] = out_ref[...] + part


def kernel(x, W_uns, W_sup, b_sup):
    wt = W_uns.T
    return pl.pallas_call(
        _fused_kernel,
        grid=(HIDDEN // BLK,),
        in_specs=[
            pl.BlockSpec((INPUT,), lambda i: (0,)),
            pl.BlockSpec((INPUT, BLK), lambda i: (0, i)),
            pl.BlockSpec((OUT, HIDDEN), lambda i: (0, 0)),
            pl.BlockSpec((OUT,), lambda i: (0,)),
        ],
        out_specs=pl.BlockSpec((OUT,), lambda i: (0,)),
        out_shape=jax.ShapeDtypeStruct((OUT,), jnp.float32),
    )(x, wt, W_sup, b_sup)


# speedup vs baseline: 1.1669x; 1.1669x over previous
import jax
import jax.numpy as jnp
from jax.experimental import pallas as pl

INPUT = 784
HIDDEN = 8192
OUT = 10
BLK = 2048


def _fused_kernel(x_ref, wt_ref, wsup_ref, b_ref, out_ref):
    i = pl.program_id(0)
    x2 = x_ref[...].reshape(1, INPUT)
    h = jax.lax.dot_general(
        x2, wt_ref[...],
        (((1,), (0,)), ((), ())),
        preferred_element_type=jnp.float32,
    )
    h = jnp.maximum(h, 0.0)
    part = jax.lax.dot_general(
        h, wsup_ref[:, pl.ds(i * BLK, BLK)],
        (((1,), (1,)), ((), ())),
        preferred_element_type=jnp.float32,
    ).reshape(OUT)

    @pl.when(i == 0)
    def _():
        out_ref[...] = b_ref[...] + part

    @pl.when(i != 0)
    def _():
        out_ref[...] = out_ref[...] + part


def kernel(x, W_uns, W_sup, b_sup):
    wt = W_uns.T
    return pl.pallas_call(
        _fused_kernel,
        grid=(HIDDEN // BLK,),
        in_specs=[
            pl.BlockSpec((INPUT,), lambda i: (0,)),
            pl.BlockSpec((INPUT, BLK), lambda i: (0, i)),
            pl.BlockSpec((OUT, HIDDEN), lambda i: (0, 0)),
            pl.BlockSpec((OUT,), lambda i: (0,)),
        ],
        out_specs=pl.BlockSpec((OUT,), lambda i: (0,)),
        out_shape=jax.ShapeDtypeStruct((OUT,), jnp.float32),
    )(x, wt, W_sup, b_sup)
